# Initial kernel scaffold; baseline (speedup 1.0000x reference)
import jax
import jax.numpy as jnp
from jax.experimental import pallas as pl


def kernel(token_ids, emb):
    raise NotImplementedError("write your pallas kernel here")


# speedup vs baseline: 1.5631x; 1.5631x over previous
import jax
import jax.numpy as jnp
from jax import lax
from jax.experimental import pallas as pl
from jax.experimental.pallas import tpu as pltpu
from jax.experimental.pallas import tpu_sc as plsc

EMBEDDING_DIM = 32
B_FLAT = 16384 * 26
NUM_CORES = 2
NUM_SUBCORES = 16
NUM_WORKERS = NUM_CORES * NUM_SUBCORES
B_PER_W = B_FLAT // NUM_WORKERS
CHUNK = 1664
N_CHUNKS = B_PER_W // CHUNK

_mesh = plsc.VectorSubcoreMesh(core_axis_name="c", subcore_axis_name="s")


def _gather_body(table_hbm, idx_hbm, out_hbm, idx_v, rows_v, sem):
    wid = lax.axis_index("s") * NUM_CORES + lax.axis_index("c")
    base = wid * B_PER_W

    def body(g, carry):
        off = base + g * CHUNK
        pltpu.sync_copy(idx_hbm.at[pl.ds(off, CHUNK)], idx_v)
        pltpu.async_copy(table_hbm.at[idx_v], rows_v, sem).wait()
        pltpu.sync_copy(rows_v, out_hbm.at[pl.ds(off, CHUNK)])
        return carry

    lax.fori_loop(0, N_CHUNKS, body, 0)


@jax.jit
def _embed(idx_flat, emb):
    k = pl.kernel(
        _gather_body,
        mesh=_mesh,
        out_type=jax.ShapeDtypeStruct((B_FLAT, EMBEDDING_DIM), jnp.float32),
        scratch_types=[
            pltpu.VMEM((CHUNK,), jnp.int32),
            pltpu.VMEM((CHUNK, EMBEDDING_DIM), jnp.float32),
            pltpu.SemaphoreType.DMA,
        ],
        compiler_params=pltpu.CompilerParams(use_tc_tiling_on_sc=False),
    )
    return k(emb, idx_flat)


def kernel(token_ids, emb):
    idx_flat = token_ids.reshape(-1)
    out = _embed(idx_flat, emb)
    return out.reshape(token_ids.shape + (EMBEDDING_DIM,))


# speedup vs baseline: 1.5673x; 1.0027x over previous
import jax
import jax.numpy as jnp
from jax import lax
from jax.experimental import pallas as pl
from jax.experimental.pallas import tpu as pltpu
from jax.experimental.pallas import tpu_sc as plsc

EMBEDDING_DIM = 32
B_FLAT = 16384 * 26
NUM_CORES = 2
NUM_SUBCORES = 16
NUM_WORKERS = NUM_CORES * NUM_SUBCORES
B_PER_W = B_FLAT // NUM_WORKERS
CHUNK = 1664
N_CHUNKS = B_PER_W // CHUNK

_mesh = plsc.VectorSubcoreMesh(core_axis_name="c", subcore_axis_name="s")


def _gather_body(table_hbm, idx_hbm, out_hbm, idx_v, rows_v, sem_g, sem_w):
    wid = lax.axis_index("s") * NUM_CORES + lax.axis_index("c")
    base = wid * B_PER_W

    pltpu.sync_copy(idx_hbm.at[pl.ds(base, B_PER_W)], idx_v)

    def gather(g, buf):
        return pltpu.async_copy(
            table_hbm.at[idx_v.at[pl.ds(g * CHUNK, CHUNK)]],
            rows_v.at[buf],
            sem_g,
        )

    writes = [None] * N_CHUNKS
    gathers = [None] * N_CHUNKS
    gathers[0] = gather(0, 0)
    for g in range(N_CHUNKS):
        buf = g % 2
        gathers[g].wait()
        writes[g] = pltpu.async_copy(
            rows_v.at[buf], out_hbm.at[pl.ds(base + g * CHUNK, CHUNK)], sem_w
        )
        if g + 1 < N_CHUNKS:
            if g >= 1:
                writes[g - 1].wait()
            gathers[g + 1] = gather(g + 1, 1 - buf)
    writes[N_CHUNKS - 2].wait()
    writes[N_CHUNKS - 1].wait()


@jax.jit
def _embed(idx_flat, emb):
    k = pl.kernel(
        _gather_body,
        mesh=_mesh,
        out_type=jax.ShapeDtypeStruct((B_FLAT, EMBEDDING_DIM), jnp.float32),
        scratch_types=[
            pltpu.VMEM((B_PER_W,), jnp.int32),
            pltpu.VMEM((2, CHUNK, EMBEDDING_DIM), jnp.float32),
            pltpu.SemaphoreType.DMA,
            pltpu.SemaphoreType.DMA,
        ],
        compiler_params=pltpu.CompilerParams(use_tc_tiling_on_sc=False),
    )
    return k(emb, idx_flat)


def kernel(token_ids, emb):
    idx_flat = token_ids.reshape(-1)
    out = _embed(idx_flat, emb)
    return out.reshape(token_ids.shape + (EMBEDDING_DIM,))
